# baseline (speedup 1.0000x reference)
import jax
import jax.numpy as jnp
from jax.experimental import pallas as pl
from jax.experimental.pallas import tpu as pltpu

_TEMP = 100.0
_BN_EPS = 1e-5
_SELU_ALPHA = 1.6732632423543772
_SELU_SCALE = 1.0507009873554805

_GAT_KEYS = (
    'pt1_w', 'pt1_b', 'pt2_w', 'pt2_b', 'ap_wT', 'ap_b',
    'w11', 'w22', 'w12', 'wM', 'apM_w', 'apM_b',
    'pwa_w', 'pwa_b', 'pwo_w', 'pwo_b',
    'pwaM_w', 'pwaM_b', 'pwoM_w', 'pwoM_b', 'bn_g', 'bn_b',
)


def _prep_gat(p):
    return [
        p['pt1_w'], p['pt1_b'].reshape(1, -1),
        p['pt2_w'], p['pt2_b'].reshape(1, -1),
        p['ap_w'].T, p['ap_b'].reshape(1, -1),
        p['w11'].reshape(1, -1), p['w22'].reshape(1, -1),
        p['w12'].reshape(1, -1), p['wM'].reshape(1, -1),
        p['apM_w'], p['apM_b'].reshape(1, -1),
        p['pwa_w'], p['pwa_b'].reshape(1, -1),
        p['pwo_w'], p['pwo_b'].reshape(1, -1),
        p['pwaM_w'], p['pwaM_b'].reshape(1, -1),
        p['pwoM_w'], p['pwoM_b'].reshape(1, -1),
        p['bn_g'].reshape(1, -1), p['bn_b'].reshape(1, -1),
    ]


def _gat_layer(x1, x2, mast, p, n1):
    x1p = x1 @ p['pt1_w'] + p['pt1_b']
    x2p = x2 @ p['pt2_w'] + p['pt2_b']
    x = jnp.concatenate([x1p, x2p], axis=0)
    n = x.shape[0]
    n2 = n - n1
    dout = p['ap_b'].shape[1]
    xt = x.T

    q = n // 4
    if q >= 128 and n1 == 2 * q:
        a0a = jnp.zeros((q, 2 * q), jnp.float32)
        a0b = jnp.zeros((q, 2 * q), jnp.float32)
        a1a = jnp.zeros((q, q), jnp.float32)
        a1b = jnp.zeros((q, 2 * q), jnp.float32)
        a2 = jnp.zeros((q, 2 * q), jnp.float32)
        a3 = jnp.zeros((q, q), jnp.float32)
        for d in range(0, dout, 2):
            ts = []
            for e in (d, d + 1):
                wrow = p['ap_wT'][e:e + 1, :]
                xd = x * wrow
                bd = p['ap_b'][0, e]
                ts.append((
                    jnp.tanh(xd[0:q] @ xt + bd),
                    jnp.tanh(xd[q:2 * q] @ xt[:, q:] + bd),
                    jnp.tanh(xd[2 * q:3 * q] @ xt[:, 2 * q:] + bd),
                    jnp.tanh(xd[3 * q:] @ xt[:, 3 * q:] + bd),
                    p['w11'][0, e], p['w12'][0, e], p['w22'][0, e]))
            (t0a, t1a, t2a, t3a, wa11, wa12, wa22), \
                (t0b, t1b, t2b, t3b, wb11, wb12, wb22) = ts
            a0a = a0a + (wa11 * t0a[:, :2 * q] + wb11 * t0b[:, :2 * q])
            a0b = a0b + (wa12 * t0a[:, 2 * q:] + wb12 * t0b[:, 2 * q:])
            a1a = a1a + (wa11 * t1a[:, :q] + wb11 * t1b[:, :q])
            a1b = a1b + (wa12 * t1a[:, q:] + wb12 * t1b[:, q:])
            a2 = a2 + (wa22 * t2a + wb22 * t2b)
            a3 = a3 + (wa22 * t3a + wb22 * t3b)
        r0 = jnp.concatenate([a0a, a0b], axis=1)
        r1 = jnp.concatenate([a0a[:, q:].T, a1a, a1b], axis=1)
        r2 = jnp.concatenate([a0b[:, :q].T, a1b[:, :q].T, a2], axis=1)
        r3 = jnp.concatenate([a0b[:, q:].T, a1b[:, q:].T, a2[:, q:].T, a3],
                             axis=1)
        board = jnp.concatenate([r0, r1, r2, r3], axis=0)
    else:
        b11 = jnp.zeros((n1, n1), jnp.float32)
        b12 = jnp.zeros((n1, n2), jnp.float32)
        b22 = jnp.zeros((n2, n2), jnp.float32)
        for d in range(0, dout, 2):
            ts = []
            for e in (d, d + 1):
                wrow = p['ap_wT'][e:e + 1, :]
                xd = x * wrow
                bd = p['ap_b'][0, e]
                ts.append((jnp.tanh(xd[:n1] @ xt + bd),
                           jnp.tanh(xd[n1:] @ xt[:, n1:] + bd),
                           p['w11'][0, e], p['w12'][0, e], p['w22'][0, e]))
            (tta, tba, wa11, wa12, wa22), (ttb, tbb_, wb11, wb12, wb22) = ts
            b11 = b11 + (wa11 * tta[:, :n1] + wb11 * ttb[:, :n1])
            b12 = b12 + (wa12 * tta[:, n1:] + wb12 * ttb[:, n1:])
            b22 = b22 + (wa22 * tba + wb22 * tbb_)
        btop = jnp.concatenate([b11, b12], axis=1)
        board = jnp.concatenate(
            [btop, jnp.concatenate([b12.T, b22], axis=1)], axis=0)

    e = jnp.exp(board * (1.0 / _TEMP))
    att = e / jnp.sum(e, axis=1, keepdims=True)
    h = (att @ x) @ p['pwa_w'] + p['pwa_b'] + x @ p['pwo_w'] + p['pwo_b']
    h = h * (p['bn_g'] * (1.0 / jnp.sqrt(1.0 + _BN_EPS))) + p['bn_b']
    h = _SELU_SCALE * jnp.where(h > 0, h, _SELU_ALPHA * (jnp.exp(h) - 1.0))

    tm = jnp.tanh((x * mast) @ p['apM_w'] + p['apM_b'])
    ml = jnp.sum(tm * p['wM'], axis=1, keepdims=True) * (1.0 / _TEMP)
    mmx = jnp.max(ml, axis=0, keepdims=True)
    me = jnp.exp(ml - mmx)
    ma = me / jnp.sum(me, axis=0, keepdims=True)
    mvec = jnp.sum(ma * x, axis=0, keepdims=True)
    mnew = mvec @ p['pwaM_w'] + p['pwaM_b'] + mast @ p['pwoM_w'] + p['pwoM_b']
    return h[:n1], h[n1:], mnew


def _pool(h, pw, pb, k):
    n = h.shape[0]
    s = jax.nn.sigmoid(h @ pw + pb)
    srow = s.T
    ii = jax.lax.broadcasted_iota(jnp.int32, (n, 1), 0)
    jj = jax.lax.broadcasted_iota(jnp.int32, (1, n), 1)
    beats = jnp.where((s > srow) | ((s == srow) & (ii < jj)), 1.0, 0.0)
    rank = jnp.sum(beats, axis=0, keepdims=True)
    rk = jax.lax.broadcasted_iota(jnp.int32, (k, 1), 0).astype(jnp.float32)
    perm = jnp.where(rank == rk, 1.0, 0.0)
    return perm @ (h * s)


def _fused_body(*args):
    g1 = {k: r[...] for k, r in zip(_GAT_KEYS, args[3:25])}
    g2 = {k: r[...] for k, r in zip(_GAT_KEYS, args[25:47])}
    ptw, ptb = args[47][...], args[48][...]
    psw, psb = args[49][...], args[50][...]
    t_ref, s_ref, mo_ref = args[51], args[52], args[53]
    x1 = args[0][0]
    x2 = args[1][0]
    mast = args[2][0]

    t1, s1, m1 = _gat_layer(x1, x2, mast, g1, x1.shape[0])
    s1p = _pool(s1, psw, psb, max(s1.shape[0] // 2, 1))
    t1p = _pool(t1, ptw, ptb, max(t1.shape[0] // 2, 1))
    t2, s2, m2 = _gat_layer(t1p, s1p, m1, g2, t1p.shape[0])
    t_ref[0] = t1p + t2
    s_ref[0] = s1p + s2
    mo_ref[0] = m1 + m2


def kernel(out_T, out_S, master, params):
    b, nt, din = out_T.shape
    ns = out_S.shape[1]
    dout = params['gat1']['ap_w'].shape[1]
    kt = max(nt // 2, 1)
    ks = max(ns // 2, 1)

    ins = ([out_T, out_S, master]
           + _prep_gat(params['gat1']) + _prep_gat(params['gat2'])
           + [params['pool_hT']['proj_w'],
              params['pool_hT']['proj_b'].reshape(1, 1),
              params['pool_hS']['proj_w'],
              params['pool_hS']['proj_b'].reshape(1, 1)])

    batch3 = lambda shape: pl.BlockSpec(shape, lambda i: (i, 0, 0))
    fixed2 = lambda shape: pl.BlockSpec(shape, lambda i: (0, 0))
    in_specs = [batch3((1, nt, din)), batch3((1, ns, din)),
                batch3((1, 1, din))]
    in_specs += [fixed2(a.shape) for a in ins[3:]]

    out_shape = (jax.ShapeDtypeStruct((b, kt, dout), jnp.float32),
                 jax.ShapeDtypeStruct((b, ks, dout), jnp.float32),
                 jax.ShapeDtypeStruct((b, 1, dout), jnp.float32))
    out_specs = (batch3((1, kt, dout)), batch3((1, ks, dout)),
                 batch3((1, 1, dout)))

    outs = pl.pallas_call(
        _fused_body,
        grid=(b,),
        in_specs=in_specs,
        out_specs=out_specs,
        out_shape=out_shape,
        compiler_params=pltpu.CompilerParams(
            dimension_semantics=("parallel",)),
    )(*ins)
    return tuple(outs)


# speedup vs baseline: 1.0726x; 1.0726x over previous
import jax
import jax.numpy as jnp
from jax.experimental import pallas as pl
from jax.experimental.pallas import tpu as pltpu

_TEMP = 100.0
_BN_EPS = 1e-5
_SELU_ALPHA = 1.6732632423543772
_SELU_SCALE = 1.0507009873554805

_GAT_KEYS = (
    'pt1_w', 'pt1_b', 'pt2_w', 'pt2_b', 'ap_wT', 'ap_b',
    'w11', 'w22', 'w12', 'wM', 'apM_w', 'apM_b',
    'pwa_w', 'pwa_b', 'pwo_w', 'pwo_b',
    'pwaM_w', 'pwaM_b', 'pwoM_w', 'pwoM_b', 'bn_g', 'bn_b',
)


def _prep_gat(p):
    return [
        p['pt1_w'], p['pt1_b'].reshape(1, -1),
        p['pt2_w'], p['pt2_b'].reshape(1, -1),
        p['ap_w'].T, p['ap_b'].reshape(1, -1),
        p['w11'].reshape(1, -1), p['w22'].reshape(1, -1),
        p['w12'].reshape(1, -1), p['wM'].reshape(1, -1),
        p['apM_w'], p['apM_b'].reshape(1, -1),
        p['pwa_w'], p['pwa_b'].reshape(1, -1),
        p['pwo_w'], p['pwo_b'].reshape(1, -1),
        p['pwaM_w'], p['pwaM_b'].reshape(1, -1),
        p['pwoM_w'], p['pwoM_b'].reshape(1, -1),
        p['bn_g'].reshape(1, -1), p['bn_b'].reshape(1, -1),
    ]


def _gat_layer(x1, x2, mast, p, n1):
    x1p = x1 @ p['pt1_w'] + p['pt1_b']
    x2p = x2 @ p['pt2_w'] + p['pt2_b']
    x = jnp.concatenate([x1p, x2p], axis=0)
    n = x.shape[0]
    n2 = n - n1
    dout = p['ap_b'].shape[1]
    xt = x.T

    q = n // 4
    if q >= 128 and n1 == 2 * q:
        a0a = jnp.zeros((q, 2 * q), jnp.float32)
        a0b = jnp.zeros((q, 2 * q), jnp.float32)
        a1a = jnp.zeros((q, q), jnp.float32)
        a1b = jnp.zeros((q, 2 * q), jnp.float32)
        a2 = jnp.zeros((q, 2 * q), jnp.float32)
        a3 = jnp.zeros((q, q), jnp.float32)
        for d in range(dout):
            wrow = p['ap_wT'][d:d + 1, :]
            xd = x * wrow
            bd = p['ap_b'][0, d]
            t0 = jnp.tanh(xd[0:q] @ xt + bd)
            t1 = jnp.tanh(xd[q:2 * q] @ xt[:, q:] + bd)
            t2 = jnp.tanh(xd[2 * q:3 * q] @ xt[:, 2 * q:] + bd)
            t3 = jnp.tanh(xd[3 * q:] @ xt[:, 3 * q:] + bd)
            w11d = p['w11'][0, d]
            w12d = p['w12'][0, d]
            w22d = p['w22'][0, d]
            a0a = a0a + w11d * t0[:, :2 * q]
            a0b = a0b + w12d * t0[:, 2 * q:]
            a1a = a1a + w11d * t1[:, :q]
            a1b = a1b + w12d * t1[:, q:]
            a2 = a2 + w22d * t2
            a3 = a3 + w22d * t3
        r0 = jnp.concatenate([a0a, a0b], axis=1)
        r1 = jnp.concatenate([a0a[:, q:].T, a1a, a1b], axis=1)
        r2 = jnp.concatenate([a0b[:, :q].T, a1b[:, :q].T, a2], axis=1)
        r3 = jnp.concatenate([a0b[:, q:].T, a1b[:, q:].T, a2[:, q:].T, a3],
                             axis=1)
        board = jnp.concatenate([r0, r1, r2, r3], axis=0)
    else:
        b11 = jnp.zeros((n1, n1), jnp.float32)
        b12 = jnp.zeros((n1, n2), jnp.float32)
        b22 = jnp.zeros((n2, n2), jnp.float32)
        for d in range(dout):
            wrow = p['ap_wT'][d:d + 1, :]
            xd = x * wrow
            mtop = xd[:n1] @ xt
            mbb = xd[n1:] @ xt[:, n1:]
            ttop = jnp.tanh(mtop + p['ap_b'][0, d])
            tbb = jnp.tanh(mbb + p['ap_b'][0, d])
            b11 = b11 + p['w11'][0, d] * ttop[:, :n1]
            b12 = b12 + p['w12'][0, d] * ttop[:, n1:]
            b22 = b22 + p['w22'][0, d] * tbb
        btop = jnp.concatenate([b11, b12], axis=1)
        board = jnp.concatenate(
            [btop, jnp.concatenate([b12.T, b22], axis=1)], axis=0)

    e = jnp.exp(board * (1.0 / _TEMP))
    att = e / jnp.sum(e, axis=1, keepdims=True)
    h = (att @ x) @ p['pwa_w'] + p['pwa_b'] + x @ p['pwo_w'] + p['pwo_b']
    h = h * (p['bn_g'] * (1.0 / jnp.sqrt(1.0 + _BN_EPS))) + p['bn_b']
    h = _SELU_SCALE * jnp.where(h > 0, h, _SELU_ALPHA * (jnp.exp(h) - 1.0))

    tm = jnp.tanh((x * mast) @ p['apM_w'] + p['apM_b'])
    ml = jnp.sum(tm * p['wM'], axis=1, keepdims=True) * (1.0 / _TEMP)
    mmx = jnp.max(ml, axis=0, keepdims=True)
    me = jnp.exp(ml - mmx)
    ma = me / jnp.sum(me, axis=0, keepdims=True)
    mvec = jnp.sum(ma * x, axis=0, keepdims=True)
    mnew = mvec @ p['pwaM_w'] + p['pwaM_b'] + mast @ p['pwoM_w'] + p['pwoM_b']
    return h[:n1], h[n1:], mnew


def _pool(h, pw, pb, k):
    n = h.shape[0]
    s = jax.nn.sigmoid(h @ pw + pb)
    srow = s.T
    ii = jax.lax.broadcasted_iota(jnp.int32, (n, 1), 0)
    jj = jax.lax.broadcasted_iota(jnp.int32, (1, n), 1)
    beats = jnp.where((s > srow) | ((s == srow) & (ii < jj)), 1.0, 0.0)
    rank = jnp.sum(beats, axis=0, keepdims=True)
    rk = jax.lax.broadcasted_iota(jnp.int32, (k, 1), 0).astype(jnp.float32)
    perm = jnp.where(rank == rk, 1.0, 0.0)
    return perm @ (h * s)


def _fused_body(*args):
    g1 = {k: r[...] for k, r in zip(_GAT_KEYS, args[3:25])}
    g2 = {k: r[...] for k, r in zip(_GAT_KEYS, args[25:47])}
    ptw, ptb = args[47][...], args[48][...]
    psw, psb = args[49][...], args[50][...]
    t_ref, s_ref, mo_ref = args[51], args[52], args[53]
    x1 = args[0][0]
    x2 = args[1][0]
    mast = args[2][0]

    t1, s1, m1 = _gat_layer(x1, x2, mast, g1, x1.shape[0])
    s1p = _pool(s1, psw, psb, max(s1.shape[0] // 2, 1))
    t1p = _pool(t1, ptw, ptb, max(t1.shape[0] // 2, 1))
    t2, s2, m2 = _gat_layer(t1p, s1p, m1, g2, t1p.shape[0])
    t_ref[0] = t1p + t2
    s_ref[0] = s1p + s2
    mo_ref[0] = m1 + m2


def kernel(out_T, out_S, master, params):
    b, nt, din = out_T.shape
    ns = out_S.shape[1]
    dout = params['gat1']['ap_w'].shape[1]
    kt = max(nt // 2, 1)
    ks = max(ns // 2, 1)

    ins = ([out_T, out_S, master]
           + _prep_gat(params['gat1']) + _prep_gat(params['gat2'])
           + [params['pool_hT']['proj_w'],
              params['pool_hT']['proj_b'].reshape(1, 1),
              params['pool_hS']['proj_w'],
              params['pool_hS']['proj_b'].reshape(1, 1)])

    batch3 = lambda shape: pl.BlockSpec(shape, lambda i: (i, 0, 0))
    fixed2 = lambda shape: pl.BlockSpec(shape, lambda i: (0, 0))
    in_specs = [batch3((1, nt, din)), batch3((1, ns, din)),
                batch3((1, 1, din))]
    in_specs += [fixed2(a.shape) for a in ins[3:]]

    out_shape = (jax.ShapeDtypeStruct((b, kt, dout), jnp.float32),
                 jax.ShapeDtypeStruct((b, ks, dout), jnp.float32),
                 jax.ShapeDtypeStruct((b, 1, dout), jnp.float32))
    out_specs = (batch3((1, kt, dout)), batch3((1, ks, dout)),
                 batch3((1, 1, dout)))

    outs = pl.pallas_call(
        _fused_body,
        grid=(b,),
        in_specs=in_specs,
        out_specs=out_specs,
        out_shape=out_shape,
        compiler_params=pltpu.CompilerParams(
            dimension_semantics=("parallel",)),
    )(*ins)
    return tuple(outs)


# speedup vs baseline: 1.0838x; 1.0104x over previous
import jax
import jax.numpy as jnp
from jax.experimental import pallas as pl
from jax.experimental.pallas import tpu as pltpu

_TEMP = 100.0
_BN_EPS = 1e-5
_SELU_ALPHA = 1.6732632423543772
_SELU_SCALE = 1.0507009873554805

_GAT_KEYS = (
    'pt1_w', 'pt1_b', 'pt2_w', 'pt2_b', 'ap_wT', 'ap_b',
    'w11', 'w22', 'w12', 'wM', 'apM_w', 'apM_b',
    'pwa_w', 'pwa_b', 'pwo_w', 'pwo_b',
    'pwaM_w', 'pwaM_b', 'pwoM_w', 'pwoM_b', 'bn_g', 'bn_b',
)


def _prep_gat(p):
    return [
        p['pt1_w'], p['pt1_b'].reshape(1, -1),
        p['pt2_w'], p['pt2_b'].reshape(1, -1),
        p['ap_w'].T, p['ap_b'].reshape(1, -1),
        p['w11'].reshape(1, -1), p['w22'].reshape(1, -1),
        p['w12'].reshape(1, -1), p['wM'].reshape(1, -1),
        p['apM_w'], p['apM_b'].reshape(1, -1),
        p['pwa_w'], p['pwa_b'].reshape(1, -1),
        p['pwo_w'], p['pwo_b'].reshape(1, -1),
        p['pwaM_w'], p['pwaM_b'].reshape(1, -1),
        p['pwoM_w'], p['pwoM_b'].reshape(1, -1),
        p['bn_g'].reshape(1, -1), p['bn_b'].reshape(1, -1),
    ]


def _gat_layer(x1, x2, mast, p, n1):
    x1p = x1 @ p['pt1_w'] + p['pt1_b']
    x2p = x2 @ p['pt2_w'] + p['pt2_b']
    x = jnp.concatenate([x1p, x2p], axis=0)
    n = x.shape[0]
    n2 = n - n1
    dout = p['ap_b'].shape[1]
    xt = x.T

    q = n // 4
    if q >= 128 and n1 == 2 * q:
        a0a = jnp.zeros((q, 2 * q), jnp.float32)
        a0b = jnp.zeros((q, 2 * q), jnp.float32)
        a1a = jnp.zeros((q, q), jnp.float32)
        a1b = jnp.zeros((q, 2 * q), jnp.float32)
        a2 = jnp.zeros((q, 2 * q), jnp.float32)
        a3 = jnp.zeros((q, q), jnp.float32)
        for d in range(dout):
            wrow = p['ap_wT'][d:d + 1, :]
            xd = x * wrow
            bd = p['ap_b'][0, d]
            t0 = jnp.tanh(xd[0:q] @ xt + bd)
            t1 = jnp.tanh(xd[q:2 * q] @ xt[:, q:] + bd)
            t2 = jnp.tanh(xd[2 * q:3 * q] @ xt[:, 2 * q:] + bd)
            t3 = jnp.tanh(xd[3 * q:] @ xt[:, 3 * q:] + bd)
            w11d = p['w11'][0, d]
            w12d = p['w12'][0, d]
            w22d = p['w22'][0, d]
            a0a = a0a + w11d * t0[:, :2 * q]
            a0b = a0b + w12d * t0[:, 2 * q:]
            a1a = a1a + w11d * t1[:, :q]
            a1b = a1b + w12d * t1[:, q:]
            a2 = a2 + w22d * t2
            a3 = a3 + w22d * t3
        r0 = jnp.concatenate([a0a, a0b], axis=1)
        r1 = jnp.concatenate([a0a[:, q:].T, a1a, a1b], axis=1)
        r2 = jnp.concatenate([a0b[:, :q].T, a1b[:, :q].T, a2], axis=1)
        r3 = jnp.concatenate([a0b[:, q:].T, a1b[:, q:].T, a2[:, q:].T, a3],
                             axis=1)
        board = jnp.concatenate([r0, r1, r2, r3], axis=0)
    else:
        b11 = jnp.zeros((n1, n1), jnp.float32)
        b12 = jnp.zeros((n1, n2), jnp.float32)
        b22 = jnp.zeros((n2, n2), jnp.float32)
        for d in range(dout):
            wrow = p['ap_wT'][d:d + 1, :]
            xd = x * wrow
            mtop = xd[:n1] @ xt
            mbb = xd[n1:] @ xt[:, n1:]
            ttop = jnp.tanh(mtop + p['ap_b'][0, d])
            tbb = jnp.tanh(mbb + p['ap_b'][0, d])
            b11 = b11 + p['w11'][0, d] * ttop[:, :n1]
            b12 = b12 + p['w12'][0, d] * ttop[:, n1:]
            b22 = b22 + p['w22'][0, d] * tbb
        btop = jnp.concatenate([b11, b12], axis=1)
        board = jnp.concatenate(
            [btop, jnp.concatenate([b12.T, b22], axis=1)], axis=0)

    e = jnp.exp(board * (1.0 / _TEMP))
    rs = jnp.sum(e, axis=1, keepdims=True)
    h = ((e @ x) @ p['pwa_w']) / rs + p['pwa_b'] + x @ p['pwo_w'] + p['pwo_b']
    h = h * (p['bn_g'] * (1.0 / jnp.sqrt(1.0 + _BN_EPS))) + p['bn_b']
    h = _SELU_SCALE * jnp.where(h > 0, h, _SELU_ALPHA * (jnp.exp(h) - 1.0))

    tm = jnp.tanh((x * mast) @ p['apM_w'] + p['apM_b'])
    ml = jnp.sum(tm * p['wM'], axis=1, keepdims=True) * (1.0 / _TEMP)
    mmx = jnp.max(ml, axis=0, keepdims=True)
    me = jnp.exp(ml - mmx)
    ma = me / jnp.sum(me, axis=0, keepdims=True)
    mvec = jnp.sum(ma * x, axis=0, keepdims=True)
    mnew = mvec @ p['pwaM_w'] + p['pwaM_b'] + mast @ p['pwoM_w'] + p['pwoM_b']
    return h[:n1], h[n1:], mnew


def _pool(h, pw, pb, k):
    n = h.shape[0]
    s = jax.nn.sigmoid(h @ pw + pb)
    srow = s.T
    ii = jax.lax.broadcasted_iota(jnp.int32, (n, 1), 0)
    jj = jax.lax.broadcasted_iota(jnp.int32, (1, n), 1)
    beats = jnp.where((s > srow) | ((s == srow) & (ii < jj)), 1.0, 0.0)
    rank = jnp.sum(beats, axis=0, keepdims=True)
    rk = jax.lax.broadcasted_iota(jnp.int32, (k, 1), 0).astype(jnp.float32)
    perm = jnp.where(rank == rk, 1.0, 0.0)
    return perm @ (h * s)


def _fused_body(*args):
    g1 = {k: r[...] for k, r in zip(_GAT_KEYS, args[3:25])}
    g2 = {k: r[...] for k, r in zip(_GAT_KEYS, args[25:47])}
    ptw, ptb = args[47][...], args[48][...]
    psw, psb = args[49][...], args[50][...]
    t_ref, s_ref, mo_ref = args[51], args[52], args[53]
    x1 = args[0][0]
    x2 = args[1][0]
    mast = args[2][0]

    t1, s1, m1 = _gat_layer(x1, x2, mast, g1, x1.shape[0])
    s1p = _pool(s1, psw, psb, max(s1.shape[0] // 2, 1))
    t1p = _pool(t1, ptw, ptb, max(t1.shape[0] // 2, 1))
    t2, s2, m2 = _gat_layer(t1p, s1p, m1, g2, t1p.shape[0])
    t_ref[0] = t1p + t2
    s_ref[0] = s1p + s2
    mo_ref[0] = m1 + m2


def kernel(out_T, out_S, master, params):
    b, nt, din = out_T.shape
    ns = out_S.shape[1]
    dout = params['gat1']['ap_w'].shape[1]
    kt = max(nt // 2, 1)
    ks = max(ns // 2, 1)

    ins = ([out_T, out_S, master]
           + _prep_gat(params['gat1']) + _prep_gat(params['gat2'])
           + [params['pool_hT']['proj_w'],
              params['pool_hT']['proj_b'].reshape(1, 1),
              params['pool_hS']['proj_w'],
              params['pool_hS']['proj_b'].reshape(1, 1)])

    batch3 = lambda shape: pl.BlockSpec(shape, lambda i: (i, 0, 0))
    fixed2 = lambda shape: pl.BlockSpec(shape, lambda i: (0, 0))
    in_specs = [batch3((1, nt, din)), batch3((1, ns, din)),
                batch3((1, 1, din))]
    in_specs += [fixed2(a.shape) for a in ins[3:]]

    out_shape = (jax.ShapeDtypeStruct((b, kt, dout), jnp.float32),
                 jax.ShapeDtypeStruct((b, ks, dout), jnp.float32),
                 jax.ShapeDtypeStruct((b, 1, dout), jnp.float32))
    out_specs = (batch3((1, kt, dout)), batch3((1, ks, dout)),
                 batch3((1, 1, dout)))

    outs = pl.pallas_call(
        _fused_body,
        grid=(b,),
        in_specs=in_specs,
        out_specs=out_specs,
        out_shape=out_shape,
        compiler_params=pltpu.CompilerParams(
            dimension_semantics=("parallel",)),
    )(*ins)
    return tuple(outs)
